# baseline (speedup 1.0000x reference)
import functools

import jax
import jax.numpy as jnp
from jax import lax
from jax.experimental import pallas as pl
from jax.experimental.pallas import tpu as pltpu
from jax.experimental.pallas import tpu_sc as plsc

R = 128
V = 100000
NW = 32
SC_END = 56320
SHARD = SC_END // NW
CB = 440
NBLK = SHARD // CB
GROUPS = R // 16
TCB = 160
TC_GRID = (V - SC_END) // TCB
NEG_INF = float("-inf")

_mesh = plsc.VectorSubcoreMesh(core_axis_name="c", subcore_axis_name="s")


@functools.partial(
    pl.kernel,
    mesh=_mesh,
    compiler_params=pltpu.CompilerParams(needs_layout_passes=False),
    out_type=(
        jax.ShapeDtypeStruct((NW, GROUPS, 16), jnp.float32),
        jax.ShapeDtypeStruct((NW, GROUPS, 16), jnp.int32),
    ),
    scratch_types=[
        pltpu.VMEM((CB, R), jnp.float32),
        pltpu.VMEM((CB, R), jnp.float32),
        pltpu.VMEM((GROUPS, 16), jnp.float32),
        pltpu.VMEM((GROUPS, 16), jnp.int32),
        pltpu.SemaphoreType.DMA,
        pltpu.SemaphoreType.DMA,
    ],
)
def _sc_argmax(xt_hbm, outv_hbm, outc_hbm, b0, b1, obufv, obufc, s0, s1):
    wid = lax.axis_index("s") * 2 + lax.axis_index("c")
    wstart = wid * SHARD
    bufs = (b0, b1)
    sems = (s0, s1)

    def blk_src(blk):
        c0 = pl.multiple_of(wstart + blk * CB, 8)
        return xt_hbm.at[pl.ds(c0, CB), pl.ds(0, R)]

    for b in range(2):
        pltpu.async_copy(blk_src(b), bufs[b], sems[b])

    neg_inf = jnp.full((16,), NEG_INF, jnp.float32)
    zero_i = jnp.zeros((16,), jnp.int32)

    def scan_block(buf, cbase, accs):
        def body(c, carry):
            acc = list(carry)
            cv = jnp.broadcast_to(cbase + c, (16,))
            for u in range(GROUPS):
                v = buf[c, pl.ds(u * 16, 16)]
                pred = v > acc[u]
                acc[u] = jnp.where(pred, v, acc[u])
                acc[GROUPS + u] = jnp.where(pred, cv, acc[GROUPS + u])
            return tuple(acc)

        return lax.fori_loop(0, CB, body, accs)

    def outer(j, carry):
        acc = carry
        for b in range(2):
            blk = j * 2 + b
            pltpu.make_async_copy(blk_src(0), bufs[b], sems[b]).wait()
            acc = scan_block(bufs[b], wstart + blk * CB, acc)

            @pl.when(blk + 2 < NBLK)
            def _():
                pltpu.async_copy(blk_src(blk + 2), bufs[b], sems[b])

        return acc

    init = tuple([neg_inf] * GROUPS + [zero_i] * GROUPS)
    acc = lax.fori_loop(0, NBLK // 2, outer, init)

    for u in range(GROUPS):
        obufv[u, pl.ds(0, 16)] = acc[u]
        obufc[u, pl.ds(0, 16)] = acc[GROUPS + u]
    pltpu.sync_copy(obufv, outv_hbm.at[wid])
    pltpu.sync_copy(obufc, outc_hbm.at[wid])


def _tc_body(x_ref, outv_ref, outc_ref, accv, acci):
    pid = pl.program_id(0)

    @pl.when(pid == 0)
    def _():
        accv[...] = jnp.full((8, R), NEG_INF, jnp.float32)
        acci[...] = jnp.zeros((8, R), jnp.int32)

    block = x_ref[...]
    av = accv[...]
    ai = acci[...]
    sub_iota = lax.broadcasted_iota(jnp.int32, (8, R), 0)
    base = SC_END + pid * TCB
    for c in range(TCB // 8):
        sub = block[c * 8:(c + 1) * 8, :]
        pred = sub > av
        av = jnp.where(pred, sub, av)
        ai = jnp.where(pred, sub_iota + (base + c * 8), ai)
    accv[...] = av
    acci[...] = ai

    @pl.when(pid == TC_GRID - 1)
    def _():
        m = jnp.max(av, axis=0, keepdims=True)
        cand = jnp.where(av == m, ai, jnp.int32(V))
        outv_ref[...] = m
        outc_ref[...] = jnp.min(cand, axis=0, keepdims=True)


_tc_argmax = pl.pallas_call(
    _tc_body,
    grid=(TC_GRID,),
    in_specs=[pl.BlockSpec((TCB, R), lambda i: (SC_END // TCB + i, 0))],
    out_specs=(
        pl.BlockSpec((1, R), lambda i: (0, 0)),
        pl.BlockSpec((1, R), lambda i: (0, 0)),
    ),
    out_shape=(
        jax.ShapeDtypeStruct((1, R), jnp.float32),
        jax.ShapeDtypeStruct((1, R), jnp.int32),
    ),
    scratch_shapes=[
        pltpu.VMEM((8, R), jnp.float32),
        pltpu.VMEM((8, R), jnp.int32),
    ],
)


def kernel(m_logits):
    xt = m_logits.T
    sc_v, sc_c = _sc_argmax(xt)
    tc_v, tc_c = _tc_argmax(xt)
    vals = jnp.concatenate([sc_v.reshape(NW, R), tc_v], axis=0)
    cols = jnp.concatenate([sc_c.reshape(NW, R), tc_c], axis=0)
    m = vals.max(axis=0)
    cand = jnp.where(vals == m[None, :], cols, jnp.int32(V))
    return cand.min(axis=0).reshape(R, 1).astype(jnp.int32)


# speedup vs baseline: 2.4920x; 2.4920x over previous
import functools

import jax
import jax.numpy as jnp
from jax import lax
from jax.experimental import pallas as pl
from jax.experimental.pallas import tpu as pltpu
from jax.experimental.pallas import tpu_sc as plsc

R = 128
V = 100000
NW = 32
SC_END = 51200
SHARD = SC_END // NW
CB = 400
NBLK = SHARD // CB
GROUPS = R // 16
TCB = 800
TC_GRID = (V - SC_END) // TCB
NEG_INF = float("-inf")

_mesh = plsc.VectorSubcoreMesh(core_axis_name="c", subcore_axis_name="s")


@functools.partial(
    pl.kernel,
    mesh=_mesh,
    compiler_params=pltpu.CompilerParams(needs_layout_passes=False),
    out_type=(
        jax.ShapeDtypeStruct((NW, GROUPS, 16), jnp.float32),
        jax.ShapeDtypeStruct((NW, GROUPS, 16), jnp.int32),
    ),
    scratch_types=[
        pltpu.VMEM((CB, R), jnp.float32),
        pltpu.VMEM((CB, R), jnp.float32),
        pltpu.VMEM((GROUPS, 16), jnp.float32),
        pltpu.VMEM((GROUPS, 16), jnp.int32),
        pltpu.SemaphoreType.DMA,
        pltpu.SemaphoreType.DMA,
    ],
)
def _sc_argmax(xt_hbm, outv_hbm, outc_hbm, b0, b1, obufv, obufc, s0, s1):
    wid = lax.axis_index("s") * 2 + lax.axis_index("c")
    wstart = wid * SHARD
    bufs = (b0, b1)
    sems = (s0, s1)

    def blk_src(blk):
        c0 = pl.multiple_of(wstart + blk * CB, 8)
        return xt_hbm.at[pl.ds(c0, CB), pl.ds(0, R)]

    for b in range(2):
        pltpu.async_copy(blk_src(b), bufs[b], sems[b])

    neg_inf = jnp.full((16,), NEG_INF, jnp.float32)
    zero_i = jnp.zeros((16,), jnp.int32)

    def scan_block(buf, cbase, accs):
        def body(c, carry):
            acc = list(carry)
            cv = jnp.broadcast_to(cbase + c, (16,))
            for u in range(GROUPS):
                v = buf[c, pl.ds(u * 16, 16)]
                pred = v > acc[u]
                acc[u] = jnp.where(pred, v, acc[u])
                acc[GROUPS + u] = jnp.where(pred, cv, acc[GROUPS + u])
            return tuple(acc)

        return lax.fori_loop(0, CB, body, accs)

    def outer(j, carry):
        acc = carry
        for b in range(2):
            blk = j * 2 + b
            pltpu.make_async_copy(blk_src(0), bufs[b], sems[b]).wait()
            acc = scan_block(bufs[b], wstart + blk * CB, acc)

            @pl.when(blk + 2 < NBLK)
            def _():
                pltpu.async_copy(blk_src(blk + 2), bufs[b], sems[b])

        return acc

    init = tuple([neg_inf] * GROUPS + [zero_i] * GROUPS)
    acc = lax.fori_loop(0, NBLK // 2, outer, init)

    for u in range(GROUPS):
        obufv[u, pl.ds(0, 16)] = acc[u]
        obufc[u, pl.ds(0, 16)] = acc[GROUPS + u]
    pltpu.sync_copy(obufv, outv_hbm.at[wid])
    pltpu.sync_copy(obufc, outc_hbm.at[wid])


def _tc_body(x_ref, outv_ref, outc_ref, accv, acci):
    pid = pl.program_id(0)

    @pl.when(pid == 0)
    def _():
        accv[...] = jnp.full((8, R), NEG_INF, jnp.float32)
        acci[...] = jnp.zeros((8, R), jnp.int32)

    block = x_ref[...]
    av = accv[...]
    ai = acci[...]
    sub_iota = lax.broadcasted_iota(jnp.int32, (8, R), 0)
    base = SC_END + pid * TCB
    for c in range(TCB // 8):
        sub = block[c * 8:(c + 1) * 8, :]
        pred = sub > av
        av = jnp.where(pred, sub, av)
        ai = jnp.where(pred, sub_iota + (base + c * 8), ai)
    accv[...] = av
    acci[...] = ai

    @pl.when(pid == TC_GRID - 1)
    def _():
        m = jnp.max(av, axis=0, keepdims=True)
        cand = jnp.where(av == m, ai, jnp.int32(V))
        outv_ref[...] = m
        outc_ref[...] = jnp.min(cand, axis=0, keepdims=True)


_tc_argmax = pl.pallas_call(
    _tc_body,
    grid=(TC_GRID,),
    in_specs=[pl.BlockSpec((TCB, R), lambda i: (SC_END // TCB + i, 0))],
    out_specs=(
        pl.BlockSpec((1, R), lambda i: (0, 0)),
        pl.BlockSpec((1, R), lambda i: (0, 0)),
    ),
    out_shape=(
        jax.ShapeDtypeStruct((1, R), jnp.float32),
        jax.ShapeDtypeStruct((1, R), jnp.int32),
    ),
    scratch_shapes=[
        pltpu.VMEM((8, R), jnp.float32),
        pltpu.VMEM((8, R), jnp.int32),
    ],
)


def kernel(m_logits):
    xt = m_logits.T
    sc_v, sc_c = _sc_argmax(xt)
    tc_v, tc_c = _tc_argmax(xt)
    vals = jnp.concatenate([sc_v.reshape(NW, R), tc_v], axis=0)
    cols = jnp.concatenate([sc_c.reshape(NW, R), tc_c], axis=0)
    m = vals.max(axis=0)
    cand = jnp.where(vals == m[None, :], cols, jnp.int32(V))
    return cand.min(axis=0).reshape(R, 1).astype(jnp.int32)


# speedup vs baseline: 3.4253x; 1.3745x over previous
import functools

import jax
import jax.numpy as jnp
from jax import lax
from jax.experimental import pallas as pl
from jax.experimental.pallas import tpu as pltpu
from jax.experimental.pallas import tpu_sc as plsc

R = 128
V = 100000
NW = 32
STRIDE = 3120
WINDOW = 3280
CB = 328
NBLK = WINDOW // CB
GROUPS = R // 16
NEG_INF = float("-inf")

_mesh = plsc.VectorSubcoreMesh(core_axis_name="c", subcore_axis_name="s")


@functools.partial(
    pl.kernel,
    mesh=_mesh,
    compiler_params=pltpu.CompilerParams(needs_layout_passes=False),
    out_type=(
        jax.ShapeDtypeStruct((NW, GROUPS, 16), jnp.float32),
        jax.ShapeDtypeStruct((NW, GROUPS, 16), jnp.int32),
    ),
    scratch_types=[
        pltpu.VMEM((CB, R), jnp.float32),
        pltpu.VMEM((CB, R), jnp.float32),
        pltpu.VMEM((CB, R), jnp.float32),
        pltpu.VMEM((GROUPS, 16), jnp.float32),
        pltpu.VMEM((GROUPS, 16), jnp.int32),
        pltpu.SemaphoreType.DMA,
        pltpu.SemaphoreType.DMA,
        pltpu.SemaphoreType.DMA,
    ],
)
def _argmax_kernel(xt_hbm, outv_hbm, outc_hbm, b0, b1, b2, obufv, obufc,
                   s0, s1, s2):
    wid = lax.axis_index("s") * 2 + lax.axis_index("c")
    wstart = wid * STRIDE
    bufs = (b0, b1, b2)
    sems = (s0, s1, s2)

    def blk_src(blk):
        c0 = pl.multiple_of(wstart + blk * CB, 8)
        return xt_hbm.at[pl.ds(c0, CB), pl.ds(0, R)]

    for b in range(3):
        pltpu.async_copy(blk_src(b), bufs[b], sems[b])

    neg_inf = jnp.full((16,), NEG_INF, jnp.float32)
    zero_i = jnp.zeros((16,), jnp.int32)

    def scan_block(buf, cbase, accs):
        def body(c, carry):
            acc = list(carry)
            cv = jnp.broadcast_to(cbase + c, (16,))
            for u in range(GROUPS):
                v = buf[c, pl.ds(u * 16, 16)]
                pred = v > acc[u]
                acc[u] = jnp.where(pred, v, acc[u])
                acc[GROUPS + u] = jnp.where(pred, cv, acc[GROUPS + u])
            return tuple(acc)

        return lax.fori_loop(0, CB, body, accs)

    def outer(j, carry):
        acc = carry
        for b in range(3):
            blk = j * 3 + b
            pltpu.make_async_copy(blk_src(0), bufs[b], sems[b]).wait()
            acc = scan_block(bufs[b], wstart + blk * CB, acc)

            @pl.when(blk + 3 < NBLK)
            def _():
                pltpu.async_copy(blk_src(blk + 3), bufs[b], sems[b])

        return acc

    init = tuple([neg_inf] * GROUPS + [zero_i] * GROUPS)
    acc = lax.fori_loop(0, (NBLK // 3), outer, init)

    pltpu.make_async_copy(blk_src(0), bufs[0], sems[0]).wait()
    acc = scan_block(bufs[0], wstart + (NBLK - 1) * CB, acc)

    for u in range(GROUPS):
        obufv[u, pl.ds(0, 16)] = acc[u]
        obufc[u, pl.ds(0, 16)] = acc[GROUPS + u]
    pltpu.sync_copy(obufv, outv_hbm.at[wid])
    pltpu.sync_copy(obufc, outc_hbm.at[wid])


def kernel(m_logits):
    outv, outc = _argmax_kernel(m_logits.T)
    vals = outv.reshape(NW, R)
    cols = outc.reshape(NW, R)
    m = vals.max(axis=0)
    cand = jnp.where(vals == m[None, :], cols, jnp.int32(V))
    return cand.min(axis=0).reshape(R, 1).astype(jnp.int32)
